# baseline (speedup 1.0000x reference)
import functools

import jax
import jax.numpy as jnp
from jax import lax
from jax.experimental import pallas as pl
from jax.experimental.pallas import tpu as pltpu
from jax.experimental.pallas import tpu_sc as plsc

N = 10000
E = 160000
NP = 10240
BN = 2048
NI = 5
K = 48
EPT = E // 16
TK = 16
NBATCH = (EPT - TK) // K
ZR = 624

_f32 = jnp.float32
_i32 = jnp.int32


def _prep_body(x_ref, w_ref, av_ref, bv_ref, hc_ref, as_ref, ad_ref):
    h = jnp.dot(x_ref[...], w_ref[...], preferred_element_type=_f32)
    hc_ref[...] = h
    pa = h @ av_ref[...]
    pb = h @ bv_ref[...]
    c = pl.program_id(1)

    @pl.when(c == 0)
    def _init():
        as_ref[...] = pa
        ad_ref[...] = pb

    @pl.when(c != 0)
    def _acc():
        as_ref[...] += pa
        ad_ref[...] += pb


def _prep(x, W, a_src, a_dst):
    din, dout = W.shape
    nc = dout // 128
    return pl.pallas_call(
        _prep_body,
        grid=(NI, nc),
        in_specs=[
            pl.BlockSpec((BN, din), lambda i, c: (i, 0)),
            pl.BlockSpec((din, 128), lambda i, c: (0, c)),
            pl.BlockSpec((128,), lambda i, c: (c,)),
            pl.BlockSpec((128,), lambda i, c: (c,)),
        ],
        out_specs=[
            pl.BlockSpec((BN, 128), lambda i, c: (c * NI + i, 0)),
            pl.BlockSpec((BN,), lambda i, c: (i,)),
            pl.BlockSpec((BN,), lambda i, c: (i,)),
        ],
        out_shape=[
            jax.ShapeDtypeStruct((nc * NP, 128), _f32),
            jax.ShapeDtypeStruct((N,), _f32),
            jax.ShapeDtypeStruct((N,), _f32),
        ],
    )(x, W, a_src, a_dst)


def _sc_edge_body(passes, hc, srce, dste, asrc, adst, accout, sout, gout,
                  asrc_v, adst_v, s_v,
                  sr0, dd0, wb0, rows0, dsc0,
                  sr1, dd1, wb1, rows1, dsc1,
                  sr2, dd2, wb2, rows2, dsc2,
                  sr_t, dd_t,
                  acc_sh, semi0, semi1, semi2, semg0, semg1, semg2,
                  sems0, sems1, sems2):
    cid = lax.axis_index("c")
    sid = lax.axis_index("s")
    zero16 = jnp.zeros((16,), _f32)
    STS = ((sr0, dd0, wb0, rows0, semi0, semg0, dsc0, sems0),
           (sr1, dd1, wb1, rows1, semi1, semg1, dsc1, sems1),
           (sr2, dd2, wb2, rows2, semi2, semg2, dsc2, sems2))

    pltpu.sync_copy(asrc, asrc_v)
    pltpu.sync_copy(adst, adst_v)

    def _mx(i, carry):
        ma, mb = carry
        av = asrc_v[pl.ds(i * 16, 16)]
        bv = adst_v[pl.ds(i * 16, 16)]
        return jnp.maximum(ma, av), jnp.maximum(mb, bv)

    neg = jnp.full((16,), -3.0e38, _f32)
    ma, mb = lax.fori_loop(0, N // 16, _mx, (neg, neg))

    def _allmax(vec):
        for sh in (8, 4, 2, 1):
            wb0[pl.ds(0, 16)] = vec
            idx = (lax.iota(_i32, 16) + sh) & 15
            vec = jnp.maximum(vec, plsc.load_gather(wb0, [idx]))
        return vec

    gs = _allmax(ma) + _allmax(mb)
    gvec = jnp.where(gs >= 0.0, gs, 0.2 * gs)
    wb0[pl.ds(0, 16)] = gvec

    @pl.when(jnp.logical_and(cid == 0, sid == 0))
    def _wg():
        pltpu.sync_copy(wb0.at[pl.ds(0, 16)], gout)

    def _zs(i, _):
        s_v[pl.ds(i * 16, 16)] = zero16
        return 0

    lax.fori_loop(0, N // 16, _zs, 0)

    def _start_idx(b, S):
        eoff = sid * EPT + b * K
        pltpu.make_async_copy(srce.at[pl.ds(eoff, K)], S[0], S[4]).start()
        pltpu.make_async_copy(dste.at[pl.ds(eoff, K)], S[1], S[4]).start()

    def _wait_idx(S):
        pltpu.make_async_copy(srce.at[pl.ds(0, K)], S[0], S[4]).wait()
        pltpu.make_async_copy(dste.at[pl.ds(0, K)], S[1], S[4]).wait()

    def _wcompute(S, cbase, gvec, do_s):
        for t in range(K // 16):
            off = t * 16
            sv = S[0][pl.ds(off, 16)]
            dv = S[1][pl.ds(off, 16)]
            av = plsc.load_gather(asrc_v, [sv])
            bv = plsc.load_gather(adst_v, [dv])
            e = av + bv
            e = jnp.where(e >= 0.0, e, 0.2 * e)
            wv = jnp.exp(e - gvec)
            S[2][pl.ds(off, 16)] = wv
            S[0][pl.ds(off, 16)] = sv + cbase
            S[6][pl.ds(off, 16)] = dv
            if do_s:
                @pl.when(cid == 0)
                def _su(dv=dv, wv=wv):
                    plsc.addupdate_scatter(s_v, [dv], wv)

    def _start_gather(S):
        pltpu.make_async_copy(hc.at[S[0]], S[3], S[5]).start()

    def _wait_gather(S):
        pltpu.make_async_copy(hc.at[S[0]], S[3], S[5]).wait()

    def _scale(S):
        def body(i, _):
            wks = [plsc.load_gather(S[2], [jnp.full((16,), 8 * i + u, _i32)])
                   for u in range(8)]
            for u in range(8):
                kk = 8 * i + u
                for j in range(8):
                    S[3][kk, pl.ds(j * 16, 16)] = (
                        S[3][kk, pl.ds(j * 16, 16)] * wks[u])
            return 0

        lax.fori_loop(0, K // 8, body, 0)

    def _scatter_start(S):
        pltpu.make_async_copy(S[3], acc_sh.at[S[6]], S[7]).start(add=True)

    def _scatter_wait(S):
        pltpu.make_async_copy(S[3], acc_sh.at[S[6]], S[7]).wait()

    for p in range(passes):
        chunk = cid * passes + p
        cbase = chunk * NP
        do_s = (p == 0)

        def _zr(i, _):
            for j in range(8):
                rows0[i, pl.ds(j * 16, 16)] = zero16
            return 0

        lax.fori_loop(0, K, _zr, 0)
        base = sid * ZR
        for t in range(ZR // K):
            pltpu.sync_copy(rows0, acc_sh.at[pl.ds(base + t * K, K)])
        if ZR % K:
            pltpu.sync_copy(rows0.at[pl.ds(0, ZR % K)],
                            acc_sh.at[pl.ds(base + (ZR // K) * K, ZR % K)])

        @pl.when(sid == 15)
        def _zlast():
            pltpu.sync_copy(rows0.at[pl.ds(0, 16)],
                            acc_sh.at[pl.ds(9984, 16)])

        plsc.subcore_barrier()

        def _fstep(bn, ci, prep_next=True, do_swait=True, drain_prev=True,
                   start_next=True):
            NXT = STS[(ci + 1) % 3]
            PRV = STS[(ci + 2) % 3]
            if prep_next:
                _wait_idx(NXT)
                if do_swait:
                    _scatter_wait(NXT)
                _wcompute(NXT, cbase, gvec, do_s)
                _start_gather(NXT)
            if drain_prev:
                _wait_gather(PRV)
                _scale(PRV)
                _scatter_start(PRV)
            if start_next:
                if isinstance(bn, int) and bn + 2 < NBATCH:
                    _start_idx(bn + 2, PRV)
                elif not isinstance(bn, int):
                    @pl.when(bn + 2 < NBATCH)
                    def _si():
                        _start_idx(bn + 2, PRV)

        _start_idx(0, STS[0])
        _start_idx(1, STS[1])
        _wait_idx(STS[0])
        _wcompute(STS[0], cbase, gvec, do_s)
        _start_gather(STS[0])
        _start_idx(2, STS[2])
        _fstep(0, 0, do_swait=False, drain_prev=False, start_next=False)
        _fstep(1, 1, do_swait=False)
        _fstep(2, 2)

        def _trip(j, _):
            _fstep(3 * j + 3, 0)
            _fstep(3 * j + 4, 1)
            _fstep(3 * j + 5, 2)
            return 0

        lax.fori_loop(0, (NBATCH - 4) // 3, _trip, 0)
        for bb in (NBATCH - 2, NBATCH - 1):
            S = STS[bb % 3]
            _wait_gather(S)
            _scale(S)
            _scatter_start(S)
        _scatter_wait(STS[(NBATCH - 3) % 3])
        _scatter_wait(STS[(NBATCH - 2) % 3])

        toff = sid * EPT + NBATCH * K
        pltpu.sync_copy(srce.at[pl.ds(toff, TK)], sr_t)
        pltpu.sync_copy(dste.at[pl.ds(toff, TK)], dd_t)
        sv = sr_t[...]
        dv = dd_t[...]
        av = plsc.load_gather(asrc_v, [sv])
        bv = plsc.load_gather(adst_v, [dv])
        et = av + bv
        et = jnp.where(et >= 0.0, et, 0.2 * et)
        wv = jnp.exp(et - gvec)
        wb0[pl.ds(0, 16)] = wv
        sr_t[...] = sv + cbase
        if do_s:
            @pl.when(cid == 0)
            def _sut():
                plsc.addupdate_scatter(s_v, [dv], wv)
        _scatter_wait(STS[(NBATCH - 1) % 3])
        pltpu.async_copy(hc.at[sr_t], rows0.at[pl.ds(0, TK)], semg0).wait()

        def _tsc(kk, _):
            wk = plsc.load_gather(wb0, [jnp.full((16,), kk, _i32)])
            for j in range(8):
                rows0[kk, pl.ds(j * 16, 16)] = (
                    rows0[kk, pl.ds(j * 16, 16)] * wk)
            return 0

        lax.fori_loop(0, TK, _tsc, 0)
        pltpu.sync_copy(rows0.at[pl.ds(0, TK)], acc_sh.at[dd_t], add=True)

        if do_s:
            @pl.when(cid == 0)
            def _ws():
                pltpu.sync_copy(s_v, sout.at[pl.ds(sid * N, N)])

        plsc.subcore_barrier()
        pltpu.sync_copy(acc_sh.at[pl.ds(base, ZR)],
                        accout.at[pl.ds(cbase + base, ZR)])

        @pl.when(sid == 15)
        def _wlast():
            pltpu.sync_copy(acc_sh.at[pl.ds(9984, 16)],
                            accout.at[pl.ds(cbase + 9984, 16)])

        if p + 1 < passes:
            plsc.subcore_barrier()


def _sc_edge(hc, srce, dste, asrc, adst, nc):
    passes = nc // 2
    mesh = plsc.VectorSubcoreMesh(core_axis_name="c", subcore_axis_name="s")
    bufset = [
        pltpu.VMEM((K,), _i32),
        pltpu.VMEM((K,), _i32),
        pltpu.VMEM((K,), _f32),
        pltpu.VMEM((K, 128), _f32),
        pltpu.VMEM((K,), _i32),
    ]
    sems = [pltpu.SemaphoreType.DMA] * 9
    f = pl.kernel(
        functools.partial(_sc_edge_body, passes),
        out_type=[
            jax.ShapeDtypeStruct((nc * NP, 128), _f32),
            jax.ShapeDtypeStruct((16 * N,), _f32),
            jax.ShapeDtypeStruct((16,), _f32),
        ],
        mesh=mesh,
        compiler_params=pltpu.CompilerParams(needs_layout_passes=False),
        scratch_types=[
            pltpu.VMEM((N,), _f32),
            pltpu.VMEM((N,), _f32),
            pltpu.VMEM((N,), _f32),
        ] + bufset + bufset + bufset + [
            pltpu.VMEM((TK,), _i32),
            pltpu.VMEM((TK,), _i32),
            pltpu.VMEM_SHARED((N, 128), _f32),
        ] + sems,
    )
    return f(hc, srce, dste, asrc, adst)


def _fin_body(nc, act, sp_ref, as_ref, ad_ref, g_ref, b_ref, *refs):
    accs = refs[:nc]
    hcs = refs[nc:2 * nc]
    o_ref = refs[2 * nc]
    g0 = jnp.max(g_ref[...])
    e = as_ref[...] + ad_ref[...]
    e = jnp.where(e >= 0.0, e, 0.2 * e)
    wself = jnp.exp(e - g0)
    den = jnp.sum(sp_ref[...], axis=0) + wself
    cols = [accs[c][...] + wself[:, None] * hcs[c][...] for c in range(nc)]
    out = jnp.concatenate(cols, axis=-1)
    out = out / den[:, None] + b_ref[...]
    if act:
        out = jnp.where(out >= 0.0, out, 0.01 * out)
    o_ref[...] = out


def _finalize(acc, spart, asrc, adst, g, b, hc, nc, act):
    dout = nc * 128
    in_specs = [
        pl.BlockSpec((16, BN), lambda i: (0, i)),
        pl.BlockSpec((BN,), lambda i: (i,)),
        pl.BlockSpec((BN,), lambda i: (i,)),
        pl.BlockSpec((16,), lambda i: (0,)),
        pl.BlockSpec((dout,), lambda i: (0,)),
    ]
    ops = [spart, asrc, adst, g, b]
    for c in range(nc):
        in_specs.append(pl.BlockSpec((BN, 128), lambda i, c=c: (c * NI + i, 0)))
        ops.append(acc)
    for c in range(nc):
        in_specs.append(pl.BlockSpec((BN, 128), lambda i, c=c: (c * NI + i, 0)))
        ops.append(hc)
    return pl.pallas_call(
        functools.partial(_fin_body, nc, act),
        grid=(NI,),
        in_specs=in_specs,
        out_specs=pl.BlockSpec((BN, dout), lambda i: (i, 0)),
        out_shape=jax.ShapeDtypeStruct((N, dout), _f32),
    )(*ops)


def _gat_layer(x, srce, dste, W, a_src, a_dst, b, act):
    nc = W.shape[1] // 128
    hc, asrc, adst = _prep(x, W, a_src, a_dst)
    acc, sflat, g = _sc_edge(hc, srce, dste, asrc, adst, nc)
    spart = sflat.reshape(16, N)
    return _finalize(acc, spart, asrc, adst, g, b, hc, nc, act)


def kernel(x, edge_index, W1, a1_src, a1_dst, b1, W2, a2_src, a2_dst, b2,
           W3, a3_src, a3_dst, b3, W4, a4_src, a4_dst, b4):
    srce = edge_index[0]
    dste = edge_index[1]
    h = _gat_layer(x, srce, dste, W1, a1_src, a1_dst, b1, True)
    h = _gat_layer(h, srce, dste, W2, a2_src, a2_dst, b2, True)
    h = _gat_layer(h, srce, dste, W3, a3_src, a3_dst, b3, True)
    return _gat_layer(h, srce, dste, W4, a4_src, a4_dst, b4, False)


# speedup vs baseline: 1.1677x; 1.1677x over previous
import functools

import jax
import jax.numpy as jnp
from jax import lax
from jax.experimental import pallas as pl
from jax.experimental.pallas import tpu as pltpu
from jax.experimental.pallas import tpu_sc as plsc

N = 10000
E = 160000
NP = 10240
BN = 2048
NI = 5
K = 48
EPT = E // 16
TK = 16
NBATCH = (EPT - TK) // K
ZR = 624

_f32 = jnp.float32
_i32 = jnp.int32


def _prep_body(x_ref, w_ref, av_ref, bv_ref, hc_ref, as_ref, ad_ref):
    h = jnp.dot(x_ref[...], w_ref[...], preferred_element_type=_f32)
    hc_ref[...] = h
    pa = h @ av_ref[...]
    pb = h @ bv_ref[...]
    c = pl.program_id(1)

    @pl.when(c == 0)
    def _init():
        as_ref[...] = pa
        ad_ref[...] = pb

    @pl.when(c != 0)
    def _acc():
        as_ref[...] += pa
        ad_ref[...] += pb


def _prep(x, W, a_src, a_dst):
    din, dout = W.shape
    nc = dout // 128
    return pl.pallas_call(
        _prep_body,
        grid=(NI, nc),
        in_specs=[
            pl.BlockSpec((BN, din), lambda i, c: (i, 0)),
            pl.BlockSpec((din, 128), lambda i, c: (0, c)),
            pl.BlockSpec((128,), lambda i, c: (c,)),
            pl.BlockSpec((128,), lambda i, c: (c,)),
        ],
        out_specs=[
            pl.BlockSpec((BN, 128), lambda i, c: (c * NI + i, 0)),
            pl.BlockSpec((BN,), lambda i, c: (i,)),
            pl.BlockSpec((BN,), lambda i, c: (i,)),
        ],
        out_shape=[
            jax.ShapeDtypeStruct((nc * NP, 128), _f32),
            jax.ShapeDtypeStruct((N,), _f32),
            jax.ShapeDtypeStruct((N,), _f32),
        ],
    )(x, W, a_src, a_dst)


def _sc_edge_body(passes, hc, srce, dste, asrc, adst, accout, sout, gout,
                  asrc_v, adst_v, s_v,
                  sr0, dd0, wb0, rows0, dsc0,
                  sr1, dd1, wb1, rows1, dsc1,
                  sr2, dd2, wb2, rows2, dsc2,
                  sr_t, dd_t,
                  acc_sh, semi0, semi1, semi2, semg0, semg1, semg2,
                  sems0, sems1, sems2):
    cid = lax.axis_index("c")
    sid = lax.axis_index("s")
    zero16 = jnp.zeros((16,), _f32)
    STS = ((sr0, dd0, wb0, rows0, semi0, semg0, dsc0, sems0),
           (sr1, dd1, wb1, rows1, semi1, semg1, dsc1, sems1),
           (sr2, dd2, wb2, rows2, semi2, semg2, dsc2, sems2))

    pltpu.sync_copy(asrc, asrc_v)
    pltpu.sync_copy(adst, adst_v)

    def _mx(i, carry):
        ma, mb = carry
        av = asrc_v[pl.ds(i * 16, 16)]
        bv = adst_v[pl.ds(i * 16, 16)]
        return jnp.maximum(ma, av), jnp.maximum(mb, bv)

    neg = jnp.full((16,), -3.0e38, _f32)
    ma, mb = lax.fori_loop(0, N // 16, _mx, (neg, neg))

    def _allmax(vec):
        for sh in (8, 4, 2, 1):
            wb0[pl.ds(0, 16)] = vec
            idx = (lax.iota(_i32, 16) + sh) & 15
            vec = jnp.maximum(vec, plsc.load_gather(wb0, [idx]))
        return vec

    gs = _allmax(ma) + _allmax(mb)
    gvec = jnp.where(gs >= 0.0, gs, 0.2 * gs)
    wb0[pl.ds(0, 16)] = gvec

    @pl.when(jnp.logical_and(cid == 0, sid == 0))
    def _wg():
        pltpu.sync_copy(wb0.at[pl.ds(0, 16)], gout)

    def _zs(i, _):
        s_v[pl.ds(i * 16, 16)] = zero16
        return 0

    lax.fori_loop(0, N // 16, _zs, 0)

    def _start_idx(b, S):
        eoff = sid * EPT + b * K
        pltpu.make_async_copy(srce.at[pl.ds(eoff, K)], S[0], S[4]).start()
        pltpu.make_async_copy(dste.at[pl.ds(eoff, K)], S[1], S[4]).start()

    def _wait_idx(S):
        pltpu.make_async_copy(srce.at[pl.ds(0, K)], S[0], S[4]).wait()
        pltpu.make_async_copy(dste.at[pl.ds(0, K)], S[1], S[4]).wait()

    def _wcompute(S, cbase, gvec, do_s):
        for t in range(K // 16):
            off = t * 16
            sv = S[0][pl.ds(off, 16)]
            dv = S[1][pl.ds(off, 16)]
            av = plsc.load_gather(asrc_v, [sv])
            bv = plsc.load_gather(adst_v, [dv])
            e = av + bv
            e = jnp.where(e >= 0.0, e, 0.2 * e)
            wv = jnp.exp(e - gvec)
            S[2][pl.ds(off, 16)] = wv
            S[0][pl.ds(off, 16)] = sv + cbase
            S[6][pl.ds(off, 16)] = dv
            if do_s:
                @pl.when(cid == 0)
                def _su(dv=dv, wv=wv):
                    plsc.addupdate_scatter(s_v, [dv], wv)

    def _start_gather(S):
        pltpu.make_async_copy(hc.at[S[0]], S[3], S[5]).start()

    def _wait_gather(S):
        pltpu.make_async_copy(hc.at[S[0]], S[3], S[5]).wait()

    def _scale(S):
        def body(i, _):
            wks = [plsc.load_gather(S[2], [jnp.full((16,), 8 * i + u, _i32)])
                   for u in range(8)]
            for u in range(8):
                kk = 8 * i + u
                for j in range(8):
                    S[3][kk, pl.ds(j * 16, 16)] = (
                        S[3][kk, pl.ds(j * 16, 16)] * wks[u])
            return 0

        lax.fori_loop(0, K // 8, body, 0)

    def _scatter_start(S):
        pltpu.make_async_copy(S[3], acc_sh.at[S[6]], S[7]).start(add=True)

    def _scatter_wait(S):
        pltpu.make_async_copy(S[3], acc_sh.at[S[6]], S[7]).wait()

    for p in range(passes):
        chunk = cid * passes + p
        cbase = chunk * NP
        do_s = (p == 0)

        def _zr(i, _):
            for j in range(8):
                rows0[i, pl.ds(j * 16, 16)] = zero16
            return 0

        lax.fori_loop(0, K, _zr, 0)
        base = sid * ZR
        for t in range(ZR // K):
            pltpu.sync_copy(rows0, acc_sh.at[pl.ds(base + t * K, K)])
        if ZR % K:
            pltpu.sync_copy(rows0.at[pl.ds(0, ZR % K)],
                            acc_sh.at[pl.ds(base + (ZR // K) * K, ZR % K)])

        @pl.when(sid == 15)
        def _zlast():
            pltpu.sync_copy(rows0.at[pl.ds(0, 16)],
                            acc_sh.at[pl.ds(9984, 16)])

        plsc.subcore_barrier()

        def _fstep(bn, ci, do_swait=True, start_next=True, drain_prev=True):
            CUR = STS[ci]
            PRV = STS[(ci + 2) % 3]
            _wait_idx(CUR)
            if do_swait:
                _scatter_wait(CUR)
            _wcompute(CUR, cbase, gvec, do_s)
            _start_gather(CUR)
            if drain_prev:
                _wait_gather(PRV)
                _scale(PRV)
                _scatter_start(PRV)
            if start_next:
                if isinstance(bn, int) and bn + 2 < NBATCH:
                    _start_idx(bn + 2, PRV)
                elif not isinstance(bn, int):
                    @pl.when(bn + 2 < NBATCH)
                    def _si():
                        _start_idx(bn + 2, PRV)

        _start_idx(0, STS[0])
        _start_idx(1, STS[1])
        _fstep(0, 0, do_swait=False, drain_prev=False)
        _fstep(1, 1, do_swait=False)
        _fstep(2, 2, do_swait=False)

        def _trip(j, _):
            _fstep(3 * j + 3, 0)
            _fstep(3 * j + 4, 1)
            _fstep(3 * j + 5, 2)
            return 0

        lax.fori_loop(0, (NBATCH - 4) // 3, _trip, 0)
        _fstep(NBATCH - 1, (NBATCH - 1) % 3, start_next=False)
        last = STS[(NBATCH - 1) % 3]
        _wait_gather(last)
        _scale(last)
        _scatter_start(last)
        _scatter_wait(STS[(NBATCH - 2) % 3])
        _scatter_wait(STS[(NBATCH - 3) % 3])

        toff = sid * EPT + NBATCH * K
        pltpu.sync_copy(srce.at[pl.ds(toff, TK)], sr_t)
        pltpu.sync_copy(dste.at[pl.ds(toff, TK)], dd_t)
        sv = sr_t[...]
        dv = dd_t[...]
        av = plsc.load_gather(asrc_v, [sv])
        bv = plsc.load_gather(adst_v, [dv])
        et = av + bv
        et = jnp.where(et >= 0.0, et, 0.2 * et)
        wv = jnp.exp(et - gvec)
        wb0[pl.ds(0, 16)] = wv
        sr_t[...] = sv + cbase
        if do_s:
            @pl.when(cid == 0)
            def _sut():
                plsc.addupdate_scatter(s_v, [dv], wv)
        _scatter_wait(STS[(NBATCH - 1) % 3])
        pltpu.async_copy(hc.at[sr_t], rows0.at[pl.ds(0, TK)], semg0).wait()

        def _tsc(kk, _):
            wk = plsc.load_gather(wb0, [jnp.full((16,), kk, _i32)])
            for j in range(8):
                rows0[kk, pl.ds(j * 16, 16)] = (
                    rows0[kk, pl.ds(j * 16, 16)] * wk)
            return 0

        lax.fori_loop(0, TK, _tsc, 0)
        pltpu.sync_copy(rows0.at[pl.ds(0, TK)], acc_sh.at[dd_t], add=True)

        if do_s:
            @pl.when(cid == 0)
            def _ws():
                pltpu.sync_copy(s_v, sout.at[pl.ds(sid * N, N)])

        plsc.subcore_barrier()
        pltpu.sync_copy(acc_sh.at[pl.ds(base, ZR)],
                        accout.at[pl.ds(cbase + base, ZR)])

        @pl.when(sid == 15)
        def _wlast():
            pltpu.sync_copy(acc_sh.at[pl.ds(9984, 16)],
                            accout.at[pl.ds(cbase + 9984, 16)])

        if p + 1 < passes:
            plsc.subcore_barrier()


def _sc_edge(hc, srce, dste, asrc, adst, nc):
    passes = nc // 2
    mesh = plsc.VectorSubcoreMesh(core_axis_name="c", subcore_axis_name="s")
    bufset = [
        pltpu.VMEM((K,), _i32),
        pltpu.VMEM((K,), _i32),
        pltpu.VMEM((K,), _f32),
        pltpu.VMEM((K, 128), _f32),
        pltpu.VMEM((K,), _i32),
    ]
    sems = [pltpu.SemaphoreType.DMA] * 9
    f = pl.kernel(
        functools.partial(_sc_edge_body, passes),
        out_type=[
            jax.ShapeDtypeStruct((nc * NP, 128), _f32),
            jax.ShapeDtypeStruct((16 * N,), _f32),
            jax.ShapeDtypeStruct((16,), _f32),
        ],
        mesh=mesh,
        compiler_params=pltpu.CompilerParams(needs_layout_passes=False),
        scratch_types=[
            pltpu.VMEM((N,), _f32),
            pltpu.VMEM((N,), _f32),
            pltpu.VMEM((N,), _f32),
        ] + bufset + bufset + bufset + [
            pltpu.VMEM((TK,), _i32),
            pltpu.VMEM((TK,), _i32),
            pltpu.VMEM_SHARED((N, 128), _f32),
        ] + sems,
    )
    return f(hc, srce, dste, asrc, adst)


def _fin_body(nc, act, sp_ref, as_ref, ad_ref, g_ref, b_ref, *refs):
    accs = refs[:nc]
    hcs = refs[nc:2 * nc]
    o_ref = refs[2 * nc]
    g0 = jnp.max(g_ref[...])
    e = as_ref[...] + ad_ref[...]
    e = jnp.where(e >= 0.0, e, 0.2 * e)
    wself = jnp.exp(e - g0)
    den = jnp.sum(sp_ref[...], axis=0) + wself
    cols = [accs[c][...] + wself[:, None] * hcs[c][...] for c in range(nc)]
    out = jnp.concatenate(cols, axis=-1)
    out = out / den[:, None] + b_ref[...]
    if act:
        out = jnp.where(out >= 0.0, out, 0.01 * out)
    o_ref[...] = out


def _finalize(acc, spart, asrc, adst, g, b, hc, nc, act):
    dout = nc * 128
    in_specs = [
        pl.BlockSpec((16, BN), lambda i: (0, i)),
        pl.BlockSpec((BN,), lambda i: (i,)),
        pl.BlockSpec((BN,), lambda i: (i,)),
        pl.BlockSpec((16,), lambda i: (0,)),
        pl.BlockSpec((dout,), lambda i: (0,)),
    ]
    ops = [spart, asrc, adst, g, b]
    for c in range(nc):
        in_specs.append(pl.BlockSpec((BN, 128), lambda i, c=c: (c * NI + i, 0)))
        ops.append(acc)
    for c in range(nc):
        in_specs.append(pl.BlockSpec((BN, 128), lambda i, c=c: (c * NI + i, 0)))
        ops.append(hc)
    return pl.pallas_call(
        functools.partial(_fin_body, nc, act),
        grid=(NI,),
        in_specs=in_specs,
        out_specs=pl.BlockSpec((BN, dout), lambda i: (i, 0)),
        out_shape=jax.ShapeDtypeStruct((N, dout), _f32),
    )(*ops)


def _gat_layer(x, srce, dste, W, a_src, a_dst, b, act):
    nc = W.shape[1] // 128
    hc, asrc, adst = _prep(x, W, a_src, a_dst)
    acc, sflat, g = _sc_edge(hc, srce, dste, asrc, adst, nc)
    spart = sflat.reshape(16, N)
    return _finalize(acc, spart, asrc, adst, g, b, hc, nc, act)


def kernel(x, edge_index, W1, a1_src, a1_dst, b1, W2, a2_src, a2_dst, b2,
           W3, a3_src, a3_dst, b3, W4, a4_src, a4_dst, b4):
    srce = edge_index[0]
    dste = edge_index[1]
    h = _gat_layer(x, srce, dste, W1, a1_src, a1_dst, b1, True)
    h = _gat_layer(h, srce, dste, W2, a2_src, a2_dst, b2, True)
    h = _gat_layer(h, srce, dste, W3, a3_src, a3_dst, b3, True)
    return _gat_layer(h, srce, dste, W4, a4_src, a4_dst, b4, False)
